# baseline (speedup 1.0000x reference)
import functools

import jax
import jax.numpy as jnp
from jax.experimental import pallas as pl
from jax.experimental.pallas import tpu as pltpu

B = 128
V = 100000
W = 2048
NBLK = (V + W - 1) // W


def _body(a_ref, x_ref, out_ref, m_ref, s_ref, g_ref):
    k = pl.program_id(0)

    @pl.when(k == 0)
    def _init():
        m_ref[...] = jnp.full_like(m_ref, -jnp.inf)
        s_ref[...] = jnp.zeros_like(s_ref)
        g_ref[...] = jnp.zeros_like(g_ref)

    x = x_ref[...]
    cols = k * W + jax.lax.broadcasted_iota(jnp.int32, (1, W), 1)
    valid = cols < V
    xm = jnp.where(valid, x, -jnp.inf)

    bm = jnp.max(xm, axis=1, keepdims=True)
    m_old = m_ref[...]
    m_new = jnp.maximum(m_old, bm)
    e = jnp.exp(xm - m_new)
    s_ref[...] = s_ref[...] * jnp.exp(m_old - m_new) + jnp.sum(e, axis=1, keepdims=True)
    m_ref[...] = m_new

    hit = cols == a_ref[...]
    g_ref[...] += jnp.sum(jnp.where(hit, x, 0.0), axis=1, keepdims=True)

    @pl.when(k == pl.num_programs(0) - 1)
    def _fin():
        out_ref[...] = g_ref[...] - m_ref[...] - jnp.log(s_ref[...])


@jax.jit
def kernel(prediction, action):
    action = action.astype(jnp.int32).reshape(B, 1)
    out = pl.pallas_call(
        _body,
        grid=(NBLK,),
        in_specs=[
            pl.BlockSpec((B, 1), lambda k: (0, 0)),
            pl.BlockSpec((B, W), lambda k: (0, k)),
        ],
        out_specs=pl.BlockSpec((B, 1), lambda k: (0, 0)),
        out_shape=jax.ShapeDtypeStruct((B, 1), jnp.float32),
        scratch_shapes=[
            pltpu.VMEM((B, 1), jnp.float32),
            pltpu.VMEM((B, 1), jnp.float32),
            pltpu.VMEM((B, 1), jnp.float32),
        ],
    )(action, prediction)
    return out.reshape(B)


# speedup vs baseline: 1.2373x; 1.2373x over previous
import functools

import jax
import jax.numpy as jnp
from jax.experimental import pallas as pl
from jax.experimental.pallas import tpu as pltpu

B = 128
V = 100000
W = 8192
NBLK = (V + W - 1) // W


def _body(a_ref, x_ref, out_ref, m_ref, s_ref, g_ref):
    k = pl.program_id(0)

    @pl.when(k == 0)
    def _init():
        m_ref[...] = jnp.full_like(m_ref, -jnp.inf)
        s_ref[...] = jnp.zeros_like(s_ref)
        g_ref[...] = jnp.zeros_like(g_ref)

    x = x_ref[...]
    cols = k * W + jax.lax.broadcasted_iota(jnp.int32, (1, W), 1)

    def step(xm):
        bm = jnp.max(xm, axis=1, keepdims=True)
        m_old = m_ref[...]
        m_new = jnp.maximum(m_old, bm)
        e = jnp.exp(xm - m_new)
        s_ref[...] = s_ref[...] * jnp.exp(m_old - m_new) + jnp.sum(
            e, axis=1, keepdims=True)
        m_ref[...] = m_new
        hit = cols == a_ref[...]
        g_ref[...] += jnp.sum(jnp.where(hit, x, 0.0), axis=1, keepdims=True)

    @pl.when(k < NBLK - 1)
    def _full():
        step(x)

    @pl.when(k == NBLK - 1)
    def _tail():
        step(jnp.where(cols < V, x, -jnp.inf))
        out_ref[...] = g_ref[...] - m_ref[...] - jnp.log(s_ref[...])


@jax.jit
def kernel(prediction, action):
    action = action.astype(jnp.int32).reshape(B, 1)
    out = pl.pallas_call(
        _body,
        grid=(NBLK,),
        in_specs=[
            pl.BlockSpec((B, 1), lambda k: (0, 0)),
            pl.BlockSpec((B, W), lambda k: (0, k)),
        ],
        out_specs=pl.BlockSpec((B, 1), lambda k: (0, 0)),
        out_shape=jax.ShapeDtypeStruct((B, 1), jnp.float32),
        scratch_shapes=[
            pltpu.VMEM((B, 1), jnp.float32),
            pltpu.VMEM((B, 1), jnp.float32),
            pltpu.VMEM((B, 1), jnp.float32),
        ],
    )(action, prediction)
    return out.reshape(B)
